# baseline (speedup 1.0000x reference)
import jax
import jax.numpy as jnp
from jax import lax
from jax.experimental import pallas as pl
from jax.experimental.pallas import tpu as pltpu
from jax.experimental.pallas import tpu_sc as plsc

VOCAB = 100000
HIDDEN = 768
SEG_NUM = 3
B, L = 1024, 200
N = B * L
NC, NS, LANES = 2, 16, 16
NW = NC * NS
PER_W = N // NW
CHUNK = 64
NCHUNK = PER_W // CHUNK
G = HIDDEN // LANES


def _body(token_hbm, seg_hbm, tok_tab, pos_tab, seg_tab_hbm, out_hbm,
          idx_v, segv, tokbuf, posbuf, segtab_v, sem1, sem2):
    wid = lax.axis_index("s") * NC + lax.axis_index("c")
    base = wid * PER_W
    pltpu.sync_copy(seg_tab_hbm, segtab_v)

    def chunk_body(g, carry):
        off = base + g * CHUNK
        pltpu.sync_copy(token_hbm.at[pl.ds(off, CHUNK)], idx_v)
        pltpu.sync_copy(seg_hbm.at[pl.ds(off, CHUNK)], segv)
        cp1 = pltpu.async_copy(tok_tab.at[idx_v], tokbuf, sem1)
        cp2 = pltpu.async_copy(pos_tab.at[idx_v], posbuf, sem2)
        cp1.wait()
        cp2.wait()

        def row_body(rb, rcarry):
            sv = segv[pl.ds(rb * LANES, LANES)]
            for k in range(LANES):
                s = sv[k]
                r = rb * LANES + k
                for j in range(G):
                    sl = pl.ds(j * LANES, LANES)
                    tokbuf[r, sl] = (tokbuf[r, sl] + posbuf[r, sl]
                                     + segtab_v[s, sl])
            return rcarry

        lax.fori_loop(0, CHUNK // LANES, row_body, 0)
        pltpu.sync_copy(tokbuf, out_hbm.at[pl.ds(off, CHUNK)])
        return carry

    lax.fori_loop(0, NCHUNK, chunk_body, 0)


def kernel(token, segment, token_table, pos_table, seg_table):
    tok_flat = token.reshape(-1).astype(jnp.int32)
    seg_flat = segment.reshape(-1).astype(jnp.int32)
    mesh = plsc.VectorSubcoreMesh(core_axis_name="c", subcore_axis_name="s")
    out = pl.kernel(
        _body,
        mesh=mesh,
        out_type=jax.ShapeDtypeStruct((N, HIDDEN), jnp.float32),
        scratch_types=[
            pltpu.VMEM((CHUNK,), jnp.int32),
            pltpu.VMEM((CHUNK,), jnp.int32),
            pltpu.VMEM((CHUNK, HIDDEN), jnp.float32),
            pltpu.VMEM((CHUNK, HIDDEN), jnp.float32),
            pltpu.VMEM((SEG_NUM, HIDDEN), jnp.float32),
            pltpu.SemaphoreType.DMA,
            pltpu.SemaphoreType.DMA,
        ],
    )(tok_flat, seg_flat, token_table, pos_table, seg_table)
    return out.reshape(B, L, HIDDEN)
